# baseline (speedup 1.0000x reference)
import functools

import jax
import jax.numpy as jnp
from jax import lax
from jax.experimental import pallas as pl
from jax.experimental.pallas import tpu as pltpu
from jax.experimental.pallas import tpu_sc as plsc

N_LEVELS = 256
VMIN, VMAX = -4.0, 4.0
Q_STEP = (VMAX - VMIN) / (N_LEVELS - 1)
INV_W = 1.0 / Q_STEP
BIAS = -(VMIN - Q_STEP / 2.0) * INV_W + 1.0

LANES = 16
NW = 32
NB = 259
TW = NB * LANES
TOTAL = 1 * 16 * 2048 * 2048
PER_TILE = TOTAL // NW
CHUNK = 32768
NCHUNK = PER_TILE // CHUNK
NPAIR = NCHUNK // 2
UNROLL = 16
SCALE = 1.0 / TOTAL

_mesh = plsc.VectorSubcoreMesh(core_axis_name="c", subcore_axis_name="s")


@functools.partial(
    pl.kernel,
    mesh=_mesh,
    out_type=jax.ShapeDtypeStruct((NW, TW), jnp.float32),
    scratch_types=[
        pltpu.VMEM((CHUNK,), jnp.float32),
        pltpu.VMEM((CHUNK,), jnp.float32),
        pltpu.VMEM((TW,), jnp.float32),
        pltpu.SemaphoreType.DMA,
        pltpu.SemaphoreType.DMA,
    ],
    compiler_params=pltpu.CompilerParams(needs_layout_passes=False),
)
def _hist_sc(x_hbm, out_hbm, buf0, buf1, table, sem0, sem1):
    wid = lax.axis_index("s") * 2 + lax.axis_index("c")
    base = wid * PER_TILE

    zeros16 = jnp.zeros((LANES,), jnp.float32)

    @plsc.parallel_loop(0, TW // LANES, unroll=4)
    def _zero(i):
        table[pl.ds(i * LANES, LANES)] = zeros16

    lane_f = lax.iota(jnp.int32, LANES).astype(jnp.float32) * float(NB)
    bias_v = lane_f + BIAS
    lo_v = lane_f
    hi_v = lane_f + float(N_LEVELS + 1)
    ones16 = jnp.ones((LANES,), jnp.float32)

    def process(buf):
        @plsc.parallel_loop(0, CHUNK // LANES, unroll=UNROLL)
        def _body(i):
            v = buf[pl.ds(i * LANES, LANES)]
            t = v * INV_W + bias_v
            t = jnp.minimum(jnp.maximum(t, lo_v), hi_v)
            plsc.addupdate_scatter(table, [t.astype(jnp.int32)], ones16)

    def start(g, buf, sem):
        off = pl.multiple_of(base + g * CHUNK, CHUNK)
        return pltpu.async_copy(x_hbm.at[pl.ds(off, CHUNK)], buf, sem)

    def wait(buf, sem):
        pltpu.make_async_copy(x_hbm.at[pl.ds(base, CHUNK)], buf, sem).wait()

    start(0, buf0, sem0)
    start(1, buf1, sem1)

    def pair_body(p, c):
        g = p * 2
        wait(buf0, sem0)
        process(buf0)
        start(g + 2, buf0, sem0)
        wait(buf1, sem1)
        process(buf1)
        start(g + 3, buf1, sem1)
        return c

    lax.fori_loop(0, NPAIR - 1, pair_body, 0)

    wait(buf0, sem0)
    process(buf0)
    wait(buf1, sem1)
    process(buf1)

    @plsc.parallel_loop(0, TW // LANES, unroll=4)
    def _scale(i):
        table[pl.ds(i * LANES, LANES)] = table[pl.ds(i * LANES, LANES)] * SCALE

    pltpu.sync_copy(table, out_hbm.at[wid])


def kernel(x, bins_edges):
    parts = _hist_sc(x.reshape(TOTAL))
    density = jnp.sum(parts.reshape(NW, LANES, NB), axis=(0, 1))[1 : N_LEVELS + 1]
    return (x, density)


# speedup vs baseline: 1.1645x; 1.1645x over previous
import functools

import jax
import jax.numpy as jnp
from jax import lax
from jax.experimental import pallas as pl
from jax.experimental.pallas import tpu as pltpu
from jax.experimental.pallas import tpu_sc as plsc

N_LEVELS = 256
VMIN, VMAX = -4.0, 4.0
Q_STEP = (VMAX - VMIN) / (N_LEVELS - 1)
INV_W = 1.0 / Q_STEP
BIAS = -(VMIN - Q_STEP / 2.0) * INV_W + 1.0

LANES = 16
NW = 32
NB = 259
TW = NB * LANES
TOTAL = 1 * 16 * 2048 * 2048
PER_TILE = TOTAL // NW
CHUNK = 32768
NCHUNK = PER_TILE // CHUNK
NPAIR = NCHUNK // 2
UNROLL = 16
SCALE = 1.0 / TOTAL

_mesh = plsc.VectorSubcoreMesh(core_axis_name="c", subcore_axis_name="s")


@functools.partial(
    pl.kernel,
    mesh=_mesh,
    out_type=jax.ShapeDtypeStruct((NW, TW), jnp.float32),
    scratch_types=[
        pltpu.VMEM((CHUNK,), jnp.float32),
        pltpu.VMEM((CHUNK,), jnp.float32),
        pltpu.VMEM((TW,), jnp.float32),
        pltpu.VMEM((TW,), jnp.float32),
        pltpu.SemaphoreType.DMA,
        pltpu.SemaphoreType.DMA,
    ],
    compiler_params=pltpu.CompilerParams(needs_layout_passes=False),
)
def _hist_sc(x_hbm, out_hbm, buf0, buf1, table, table2, sem0, sem1):
    wid = lax.axis_index("s") * 2 + lax.axis_index("c")
    base = wid * PER_TILE

    zeros16 = jnp.zeros((LANES,), jnp.float32)

    @plsc.parallel_loop(0, TW // LANES, unroll=4)
    def _zero(i):
        table[pl.ds(i * LANES, LANES)] = zeros16
        table2[pl.ds(i * LANES, LANES)] = zeros16

    lane_f = lax.iota(jnp.int32, LANES).astype(jnp.float32) * float(NB)
    bias_v = lane_f + BIAS
    lo_v = lane_f
    hi_v = lane_f + float(N_LEVELS + 1)
    ones16 = jnp.ones((LANES,), jnp.float32)

    def process(buf):
        @plsc.parallel_loop(0, CHUNK // (2 * LANES), unroll=UNROLL // 2)
        def _body(i):
            v0 = buf[pl.ds(i * 2 * LANES, LANES)]
            v1 = buf[pl.ds(i * 2 * LANES + LANES, LANES)]
            t0 = jnp.minimum(jnp.maximum(v0 * INV_W + bias_v, lo_v), hi_v)
            t1 = jnp.minimum(jnp.maximum(v1 * INV_W + bias_v, lo_v), hi_v)
            plsc.addupdate_scatter(table, [t0.astype(jnp.int32)], ones16)
            plsc.addupdate_scatter(table2, [t1.astype(jnp.int32)], ones16)

    def start(g, buf, sem):
        off = pl.multiple_of(base + g * CHUNK, CHUNK)
        return pltpu.async_copy(x_hbm.at[pl.ds(off, CHUNK)], buf, sem)

    def wait(buf, sem):
        pltpu.make_async_copy(x_hbm.at[pl.ds(base, CHUNK)], buf, sem).wait()

    start(0, buf0, sem0)
    start(1, buf1, sem1)

    def pair_body(p, c):
        g = p * 2
        wait(buf0, sem0)
        process(buf0)
        start(g + 2, buf0, sem0)
        wait(buf1, sem1)
        process(buf1)
        start(g + 3, buf1, sem1)
        return c

    lax.fori_loop(0, NPAIR - 1, pair_body, 0)

    wait(buf0, sem0)
    process(buf0)
    wait(buf1, sem1)
    process(buf1)

    @plsc.parallel_loop(0, TW // LANES, unroll=4)
    def _scale(i):
        s = table[pl.ds(i * LANES, LANES)] + table2[pl.ds(i * LANES, LANES)]
        table[pl.ds(i * LANES, LANES)] = s * SCALE

    pltpu.sync_copy(table, out_hbm.at[wid])


def kernel(x, bins_edges):
    parts = _hist_sc(x.reshape(TOTAL))
    density = jnp.sum(parts.reshape(NW, LANES, NB), axis=(0, 1))[1 : N_LEVELS + 1]
    return (x, density)
